# baseline (speedup 1.0000x reference)
import functools

import jax
import jax.numpy as jnp
from jax import lax
from jax.experimental import pallas as pl
from jax.experimental.pallas import tpu as pltpu
from jax.experimental.pallas import tpu_sc as plsc

B = 2
N = 10000
E = 320000
D = 128
R = B * (N + E)
NC, NS = 2, 16
NW = NC * NS
W = 96
G = 3
NWIN = R // W
WPT = -(-(-(-NWIN // NW)) // G) * G
RPAD = NW * WPT * W
SLAB = WPT * W


def _gunpool_sc(xh, idx0, idx1):
    mesh = plsc.VectorSubcoreMesh(core_axis_name="c", subcore_axis_name="s")

    @functools.partial(
        pl.kernel,
        out_type=jax.ShapeDtypeStruct((R, D), jnp.float32),
        mesh=mesh,
        scratch_types=[
            pltpu.VMEM((SLAB,), jnp.int32),
            pltpu.VMEM((SLAB,), jnp.int32),
        ]
        + [pltpu.VMEM((W, D), jnp.float32)] * (2 * G + 1)
        + [pltpu.SemaphoreType.DMA] * (2 * G + 1),
    )
    def k(x_hbm, i0_hbm, i1_hbm, out_hbm, i0_all, i1_all, *rest):
        bufs = rest[: 2 * G + 1]
        sems = rest[2 * G + 1 :]
        stb, sst = bufs[2 * G], sems[2 * G]
        slots = [
            (bufs[2 * j], bufs[2 * j + 1], sems[2 * j], sems[2 * j + 1])
            for j in range(G)
        ]

        wid = lax.axis_index("s") * NC + lax.axis_index("c")
        half = jnp.full((16,), 0.5, jnp.float32)

        pltpu.sync_copy(i0_hbm.at[pl.ds(wid * SLAB, SLAB)], i0_all)
        pltpu.sync_copy(i1_hbm.at[pl.ds(wid * SLAB, SLAB)], i1_all)

        def gwin(s):
            return s * NW + wid

        def live(s):
            return gwin(s) < NWIN

        def gather(s, j):
            d0, d1, s0, s1 = slots[j]

            @pl.when(jnp.logical_and(live(s), s < WPT))
            def _():
                pltpu.async_copy(x_hbm.at[i0_all.at[pl.ds(s * W, W)]], d0, s0)
                pltpu.async_copy(x_hbm.at[i1_all.at[pl.ds(s * W, W)]], d1, s1)

        def wait_gather(s, j):
            d0, d1, s0, s1 = slots[j]

            @pl.when(live(s))
            def _():
                pltpu.make_async_copy(
                    x_hbm.at[i0_all.at[pl.ds(s * W, W)]], d0, s0).wait()
                pltpu.make_async_copy(
                    x_hbm.at[i1_all.at[pl.ds(s * W, W)]], d1, s1).wait()

        def wait_store(s):
            pltpu.make_async_copy(
                stb, out_hbm.at[pl.ds(gwin(s) * W, W)], sst).wait()

        def accum_store(s, j):
            d0, d1, _, _ = slots[j]

            @pl.when(live(s))
            def _():
                @pl.when(s > 0)
                def _():
                    wait_store(s - 1)

                @pl.loop(0, W)
                def _(r):
                    for c in range(0, D, 16):
                        stb[r, pl.ds(c, 16)] = half * (
                            d0[r, pl.ds(c, 16)] + d1[r, pl.ds(c, 16)])

                pltpu.async_copy(stb, out_hbm.at[pl.ds(gwin(s) * W, W)], sst)

        for j in range(G):
            gather(j, j)

        @pl.loop(0, WPT // G)
        def _(it):
            base = it * G
            for j in range(G):
                s = base + j
                wait_gather(s, j)
                accum_store(s, j)
                gather(s + G, j)

        live_wins = -(-(NWIN - wid) // NW)
        wait_store(live_wins - 1)

    return k(xh, idx0, idx1)


def kernel(inputs, unpool_idx):
    u0 = unpool_idx[:, 0].astype(jnp.int32)
    u1 = unpool_idx[:, 1].astype(jnp.int32)
    ar = jnp.arange(N, dtype=jnp.int32)
    pad = jnp.zeros((RPAD - R,), jnp.int32)
    idx0 = jnp.concatenate([ar, u0, ar + N, u0 + N, pad])
    idx1 = jnp.concatenate([ar, u1, ar + N, u1 + N, pad])
    perm = lambda ix: (
        ix.reshape(WPT, NW, W).transpose(1, 0, 2).reshape(RPAD))
    xh = inputs.reshape(B * N, D)
    out = _gunpool_sc(xh, perm(idx0), perm(idx1))
    return out.reshape(B, N + E, D)


# speedup vs baseline: 1.0798x; 1.0798x over previous
import functools

import jax
import jax.numpy as jnp
from jax import lax
from jax.experimental import pallas as pl
from jax.experimental.pallas import tpu as pltpu
from jax.experimental.pallas import tpu_sc as plsc

B = 2
N = 10000
E = 320000
D = 128
R = B * (N + E)
NC, NS = 2, 16
NW = NC * NS
W = 96
G = 3
NWIN = R // W
WPT = -(-(-(-NWIN // NW)) // G) * G
RPAD = NW * WPT * W
SLAB = WPT * W


def _gunpool_sc(xh, idx0, idx1):
    mesh = plsc.VectorSubcoreMesh(core_axis_name="c", subcore_axis_name="s")

    @functools.partial(
        pl.kernel,
        out_type=jax.ShapeDtypeStruct((R, D), jnp.float32),
        mesh=mesh,
        scratch_types=[
            pltpu.VMEM((SLAB,), jnp.int32),
            pltpu.VMEM((SLAB,), jnp.int32),
        ]
        + [pltpu.VMEM((W, D), jnp.float32)] * (2 * G + 1)
        + [pltpu.SemaphoreType.DMA] * (2 * G + 1),
    )
    def k(x_hbm, i0_hbm, i1_hbm, out_hbm, i0_all, i1_all, *rest):
        bufs = rest[: 2 * G + 1]
        sems = rest[2 * G + 1 :]
        stb, sst = bufs[2 * G], sems[2 * G]
        slots = [
            (bufs[2 * j], bufs[2 * j + 1], sems[2 * j], sems[2 * j + 1])
            for j in range(G)
        ]

        wid = lax.axis_index("s") * NC + lax.axis_index("c")
        half = jnp.full((16,), 0.5, jnp.float32)
        slab = wid + (wid == 16).astype(jnp.int32) - (wid == 17).astype(jnp.int32)
        base_win = slab * WPT

        pltpu.sync_copy(i0_hbm.at[pl.ds(slab * SLAB, SLAB)], i0_all)
        pltpu.sync_copy(i1_hbm.at[pl.ds(slab * SLAB, SLAB)], i1_all)

        def gwin(s):
            return base_win + s

        def live(s):
            return gwin(s) < NWIN

        def gather(s, j):
            d0, d1, s0, s1 = slots[j]

            @pl.when(jnp.logical_and(live(s), s < WPT))
            def _():
                pltpu.async_copy(x_hbm.at[i0_all.at[pl.ds(s * W, W)]], d0, s0)
                pltpu.async_copy(x_hbm.at[i1_all.at[pl.ds(s * W, W)]], d1, s1)

        def wait_gather(s, j):
            d0, d1, s0, s1 = slots[j]

            @pl.when(live(s))
            def _():
                pltpu.make_async_copy(
                    x_hbm.at[i0_all.at[pl.ds(s * W, W)]], d0, s0).wait()
                pltpu.make_async_copy(
                    x_hbm.at[i1_all.at[pl.ds(s * W, W)]], d1, s1).wait()

        def wait_store(s):
            pltpu.make_async_copy(
                stb, out_hbm.at[pl.ds(gwin(s) * W, W)], sst).wait()

        def accum_store(s, j):
            d0, d1, _, _ = slots[j]

            @pl.when(live(s))
            def _():
                @pl.when(s > 0)
                def _():
                    wait_store(s - 1)

                @pl.loop(0, W)
                def _(r):
                    for c in range(0, D, 16):
                        stb[r, pl.ds(c, 16)] = half * (
                            d0[r, pl.ds(c, 16)] + d1[r, pl.ds(c, 16)])

                pltpu.async_copy(stb, out_hbm.at[pl.ds(gwin(s) * W, W)], sst)

        for j in range(G):
            gather(j, j)

        @pl.loop(0, WPT // G)
        def _(it):
            base = it * G
            for j in range(G):
                s = base + j
                wait_gather(s, j)
                accum_store(s, j)
                gather(s + G, j)

        live_wins = jnp.clip(NWIN - base_win, 0, WPT)
        wait_store(live_wins - 1)

    return k(xh, idx0, idx1)


def kernel(inputs, unpool_idx):
    u0 = unpool_idx[:, 0].astype(jnp.int32)
    u1 = unpool_idx[:, 1].astype(jnp.int32)
    ar = jnp.arange(N, dtype=jnp.int32)
    pad = jnp.zeros((RPAD - R,), jnp.int32)
    idx0 = jnp.concatenate([ar, u0, ar + N, u0 + N, pad])
    idx1 = jnp.concatenate([ar, u1, ar + N, u1 + N, pad])
    xh = inputs.reshape(B * N, D)
    out = _gunpool_sc(xh, idx0, idx1)
    return out.reshape(B, N + E, D)
